# Initial kernel scaffold; baseline (speedup 1.0000x reference)
import jax
import jax.numpy as jnp
from jax.experimental import pallas as pl


def kernel(positions, positional_embeddings):
    raise NotImplementedError("write your pallas kernel here")


# speedup vs baseline: 1.0312x; 1.0312x over previous
import functools

import jax
import jax.numpy as jnp
from jax import lax
from jax.experimental import pallas as pl
from jax.experimental.pallas import tpu as pltpu
from jax.experimental.pallas import tpu_sc as plsc

D = 1024
L = 16
NC, NS = 2, 16
NW = NC * NS
C = 64


def _rsqrt_vec(x):
    i = lax.bitcast_convert_type(x, jnp.int32)
    i = jnp.int32(0x5F3759DF) - (i >> 1)
    y = lax.bitcast_convert_type(i, jnp.float32)
    for _ in range(3):
        y = y * (1.5 - 0.5 * x * y * y)
    return y


def _make_sc_kernel(B):
    rows_per_w = B // NW
    nchunk = rows_per_w // C
    mesh = plsc.VectorSubcoreMesh(core_axis_name="c", subcore_axis_name="s")

    @functools.partial(
        pl.kernel,
        mesh=mesh,
        out_type=jax.ShapeDtypeStruct((B, D), jnp.float32),
        scratch_types=[
            pltpu.VMEM((rows_per_w,), jnp.int32),
            pltpu.VMEM((C, D), jnp.float32),
            pltpu.SemaphoreType.DMA,
        ],
    )
    def k(pos_hbm, table_hbm, out_hbm, idx_v, buf, sem):
        wid = lax.axis_index("s") * NC + lax.axis_index("c")
        row0 = wid * rows_per_w
        pltpu.sync_copy(pos_hbm.at[pl.ds(row0, rows_per_w)], idx_v)

        def row_body(r, carry):
            acc = jnp.zeros((L,), jnp.float32)
            for j in range(D // L):
                v = buf[r, pl.ds(j * L, L)]
                acc = acc + v * v
            lane = lax.iota(jnp.int32, L)
            for k in (8, 4, 2, 1):
                perm = jnp.bitwise_xor(lane, k)
                acc = acc + acc.at[perm].get(mode="promise_in_bounds")
            inv = _rsqrt_vec(jnp.maximum(acc, 1e-24))
            for j in range(D // L):
                buf[r, pl.ds(j * L, L)] = buf[r, pl.ds(j * L, L)] * inv
            return carry

        def chunk_body(c, carry):
            base = c * C
            pltpu.async_copy(table_hbm.at[idx_v.at[pl.ds(base, C)]], buf, sem).wait()
            lax.fori_loop(0, C, row_body, 0)
            pltpu.async_copy(buf, out_hbm.at[pl.ds(row0 + base, C)], sem).wait()
            return carry

        lax.fori_loop(0, nchunk, chunk_body, 0)

    return k


def kernel(positions, positional_embeddings):
    B = positions.size
    pos_flat = positions.reshape(-1).astype(jnp.int32)
    table = positional_embeddings.astype(jnp.float32)
    out = _make_sc_kernel(B)(pos_flat, table)
    return out.reshape(positions.shape + (D,))


# speedup vs baseline: 1.4380x; 1.3945x over previous
import functools

import jax
import jax.numpy as jnp
from jax import lax
from jax.experimental import pallas as pl
from jax.experimental.pallas import tpu as pltpu
from jax.experimental.pallas import tpu_sc as plsc

D = 1024
L = 16
NC, NS = 2, 16
NW = NC * NS
C = 16


def _rsqrt_vec(x):
    i = lax.bitcast_convert_type(x, jnp.int32)
    i = jnp.int32(0x5F3759DF) - (i >> 1)
    y = lax.bitcast_convert_type(i, jnp.float32)
    for _ in range(3):
        y = y * (1.5 - 0.5 * x * y * y)
    return y


def _make_sc_kernel(B):
    rows_per_w = B // NW
    nchunk = rows_per_w // C
    npair = nchunk // 2
    mesh = plsc.VectorSubcoreMesh(core_axis_name="c", subcore_axis_name="s")

    @functools.partial(
        pl.kernel,
        mesh=mesh,
        out_type=jax.ShapeDtypeStruct((B, D), jnp.float32),
        scratch_types=[
            pltpu.VMEM((rows_per_w,), jnp.int32),
            pltpu.VMEM((C, D), jnp.float32),
            pltpu.VMEM((C, D), jnp.float32),
            pltpu.VMEM((C, D), jnp.float32),
            pltpu.VMEM((C, D), jnp.float32),
            pltpu.SemaphoreType.DMA,
            pltpu.SemaphoreType.DMA,
            pltpu.SemaphoreType.DMA,
            pltpu.SemaphoreType.DMA,
        ],
    )
    def k(pos_hbm, table_hbm, out_hbm, idx_v, bin0, bin1, bout0, bout1,
          gs0, gs1, ss0, ss1):
        wid = lax.axis_index("s") * NC + lax.axis_index("c")
        row0 = wid * rows_per_w
        pltpu.sync_copy(pos_hbm.at[pl.ds(row0, rows_per_w)], idx_v)

        bins, bouts = (bin0, bin1), (bout0, bout1)
        gsems, ssems = (gs0, gs1), (ss0, ss1)

        def gather_start(c, b):
            pltpu.async_copy(
                table_hbm.at[idx_v.at[pl.ds(c * C, C)]], bins[b], gsems[b])

        def gather_wait(b):
            pltpu.make_async_copy(
                table_hbm.at[idx_v.at[pl.ds(0, C)]], bins[b], gsems[b]).wait()

        def scatter_start(c, b):
            pltpu.async_copy(
                bouts[b], out_hbm.at[pl.ds(row0 + c * C, C)], ssems[b])

        def scatter_wait(b):
            pltpu.make_async_copy(
                bouts[b], out_hbm.at[pl.ds(row0, C)], ssems[b]).wait()

        def compute(b):
            src, dst = bins[b], bouts[b]

            def row_body(r, carry):
                acc = jnp.zeros((L,), jnp.float32)
                for j in range(D // L):
                    v = src[r, pl.ds(j * L, L)]
                    acc = acc + v * v
                lane = lax.iota(jnp.int32, L)
                for kk in (8, 4, 2, 1):
                    perm = jnp.bitwise_xor(lane, kk)
                    acc = acc + acc.at[perm].get(mode="promise_in_bounds")
                inv = _rsqrt_vec(jnp.maximum(acc, 1e-24))
                for j in range(D // L):
                    dst[r, pl.ds(j * L, L)] = src[r, pl.ds(j * L, L)] * inv
                return carry

            lax.fori_loop(0, C, row_body, 0)

        gather_start(0, 0)
        gather_start(1, 1)

        def pair_body(p, carry):
            for b in (0, 1):
                c = 2 * p + b
                gather_wait(b)

                @pl.when(p > 0)
                def _():
                    scatter_wait(b)

                compute(b)
                scatter_start(c, b)

                @pl.when(p < npair - 1)
                def _():
                    gather_start(c + 2, b)

            return carry

        lax.fori_loop(0, npair, pair_body, 0)
        scatter_wait(0)
        scatter_wait(1)

    return k


def kernel(positions, positional_embeddings):
    B = positions.size
    pos_flat = positions.reshape(-1).astype(jnp.int32)
    table = positional_embeddings.astype(jnp.float32)
    out = _make_sc_kernel(B)(pos_flat, table)
    return out.reshape(positions.shape + (D,))


# speedup vs baseline: 1.6805x; 1.1686x over previous
import functools

import jax
import jax.numpy as jnp
from jax import lax
from jax.experimental import pallas as pl
from jax.experimental.pallas import tpu as pltpu
from jax.experimental.pallas import tpu_sc as plsc

D = 1024
L = 16
NC, NS = 2, 16
NW = NC * NS
C = 16


def _rsqrt_vec(x):
    i = lax.bitcast_convert_type(x, jnp.int32)
    i = jnp.int32(0x5F3759DF) - (i >> 1)
    y = lax.bitcast_convert_type(i, jnp.float32)
    for _ in range(3):
        y = y * (1.5 - 0.5 * x * y * y)
    return y


def _make_sc_kernel(B):
    rows_per_w = B // NW
    nchunk = rows_per_w // C
    npair = nchunk // 2
    mesh = plsc.VectorSubcoreMesh(core_axis_name="c", subcore_axis_name="s")

    @functools.partial(
        pl.kernel,
        mesh=mesh,
        out_type=jax.ShapeDtypeStruct((B, D), jnp.float32),
        scratch_types=[
            pltpu.VMEM((rows_per_w,), jnp.int32),
            pltpu.VMEM((C, D), jnp.float32),
            pltpu.VMEM((C, D), jnp.float32),
            pltpu.VMEM((C, D), jnp.float32),
            pltpu.VMEM((C, D), jnp.float32),
            pltpu.SemaphoreType.DMA,
            pltpu.SemaphoreType.DMA,
            pltpu.SemaphoreType.DMA,
            pltpu.SemaphoreType.DMA,
        ],
    )
    def k(pos_hbm, table_hbm, out_hbm, idx_v, bin0, bin1, bout0, bout1,
          gs0, gs1, ss0, ss1):
        wid = lax.axis_index("s") * NC + lax.axis_index("c")
        row0 = wid * rows_per_w
        pltpu.sync_copy(pos_hbm.at[pl.ds(row0, rows_per_w)], idx_v)

        bins, bouts = (bin0, bin1), (bout0, bout1)
        gsems, ssems = (gs0, gs1), (ss0, ss1)

        def gather_start(c, b):
            pltpu.async_copy(
                table_hbm.at[idx_v.at[pl.ds(c * C, C)]], bins[b], gsems[b])

        def gather_wait(b):
            pltpu.make_async_copy(
                table_hbm.at[idx_v.at[pl.ds(0, C)]], bins[b], gsems[b]).wait()

        def scatter_start(c, b):
            pltpu.async_copy(
                bouts[b], out_hbm.at[pl.ds(row0 + c * C, C)], ssems[b])

        def scatter_wait(b):
            pltpu.make_async_copy(
                bouts[b], out_hbm.at[pl.ds(row0, C)], ssems[b]).wait()

        def compute(b):
            src, dst = bins[b], bouts[b]

            def row_body(r, carry):
                accs = [jnp.zeros((L,), jnp.float32) for _ in range(8)]
                for j in range(D // L):
                    v = src[r, pl.ds(j * L, L)]
                    accs[j % 8] = accs[j % 8] + v * v
                acc01 = accs[0] + accs[1]
                acc23 = accs[2] + accs[3]
                acc45 = accs[4] + accs[5]
                acc67 = accs[6] + accs[7]
                acc = (acc01 + acc23) + (acc45 + acc67)
                lane = lax.iota(jnp.int32, L)
                for kk in (8, 4, 2, 1):
                    perm = jnp.bitwise_xor(lane, kk)
                    acc = acc + acc.at[perm].get(mode="promise_in_bounds")
                inv = _rsqrt_vec(jnp.maximum(acc, 1e-24))
                for j in range(D // L):
                    dst[r, pl.ds(j * L, L)] = src[r, pl.ds(j * L, L)] * inv
                return carry

            lax.fori_loop(0, C, row_body, 0)

        gather_start(0, 0)
        gather_start(1, 1)

        def pair_body(p, carry):
            for b in (0, 1):
                c = 2 * p + b
                gather_wait(b)

                @pl.when(p > 0)
                def _():
                    scatter_wait(b)

                compute(b)
                scatter_start(c, b)

                @pl.when(p < npair - 1)
                def _():
                    gather_start(c + 2, b)

            return carry

        lax.fori_loop(0, npair, pair_body, 0)
        scatter_wait(0)
        scatter_wait(1)

    return k


def kernel(positions, positional_embeddings):
    B = positions.size
    pos_flat = positions.reshape(-1).astype(jnp.int32)
    table = positional_embeddings.astype(jnp.float32)
    out = _make_sc_kernel(B)(pos_flat, table)
    return out.reshape(positions.shape + (D,))
